# baseline (speedup 1.0000x reference)
import functools

import jax
import jax.numpy as jnp
from jax import lax
from jax.experimental import pallas as pl
from jax.experimental.pallas import tpu as pltpu
from jax.experimental.pallas import tpu_sc as plsc

N = 10000
D = 128
E = 320000
NS = 16
CHUNK = 64
CH_PER_TILE = 320
E_PAD = NS * CH_PER_TILE * CHUNK
N_PAD = 10112
ROWS_PER_TILE = N_PAD // NS
NB = 4
NI = 8
UNROLL = 8


@functools.partial(
    pl.kernel,
    out_type=jax.ShapeDtypeStruct((N_PAD, D), jnp.float32),
    mesh=plsc.VectorSubcoreMesh(
        core_axis_name="c", subcore_axis_name="s", num_cores=1, num_subcores=NS
    ),
    scratch_types=[
        pltpu.VMEM((NB, CHUNK, D), jnp.float32),
        pltpu.VMEM((NI, CHUNK), jnp.int32),
        pltpu.VMEM((NI, CHUNK), jnp.int32),
        pltpu.VMEM_SHARED((N_PAD, D), jnp.float32),
        pltpu.SemaphoreType.DMA((NB,)),
        pltpu.SemaphoreType.DMA((NB,)),
        pltpu.SemaphoreType.DMA((NI,)),
        pltpu.SemaphoreType.DMA((NI,)),
    ],
)
def _sc_aggregate(x_hbm, src_hbm, dst_hbm, zeros_hbm, out_hbm,
                  rows_v, si_v, di_v, agg_sh, sg, ss, s_si, s_di):
    sid = lax.axis_index("s")
    stripe = sid * ROWS_PER_TILE
    last = CH_PER_TILE - 1

    pltpu.sync_copy(zeros_hbm, agg_sh.at[pl.ds(stripe, ROWS_PER_TILE)])
    plsc.subcore_barrier()

    def i_start(c, ib):
        pltpu.async_copy(src_hbm.at[sid, c], si_v.at[ib], s_si.at[ib])
        pltpu.async_copy(dst_hbm.at[sid, c], di_v.at[ib], s_di.at[ib])

    def si_wait(c, ib):
        pltpu.make_async_copy(src_hbm.at[sid, c], si_v.at[ib],
                              s_si.at[ib]).wait()

    def di_wait(c, ib):
        pltpu.make_async_copy(dst_hbm.at[sid, c], di_v.at[ib],
                              s_di.at[ib]).wait()

    def g_start(b, ib):
        pltpu.async_copy(x_hbm.at[si_v.at[ib]], rows_v.at[b], sg.at[b])

    def g_wait(b, ib):
        pltpu.make_async_copy(x_hbm.at[si_v.at[ib]], rows_v.at[b],
                              sg.at[b]).wait()

    def s_start(b, ib):
        pltpu.async_copy(rows_v.at[b], agg_sh.at[di_v.at[ib]], ss.at[b],
                         add=True)

    def s_wait(b, ib):
        pltpu.make_async_copy(rows_v.at[b], agg_sh.at[di_v.at[ib]],
                              ss.at[b]).wait()

    def step(j, k, skip):
        b, bm = k % NB, (k - 1) % NB
        ib, ibm, ibn = k % NI, (k - 1) % NI, (k + 2) % NI
        if skip <= -4:
            s_wait(b, (k - 4) % NI)
        i_start(jnp.minimum(j + 2, last), ibn)
        si_wait(j, ib)
        g_start(b, ib)
        if skip <= -1:
            g_wait(bm, ibm)
            di_wait(j - 1, ibm)
            s_start(bm, ibm)

    i_start(0, 0)
    i_start(1, 1)
    for j in range(UNROLL):
        step(j, j, -j)

    def body(jj, carry):
        for k in range(UNROLL):
            step(jj * UNROLL + k, k, -4)
        return carry

    lax.fori_loop(1, CH_PER_TILE // UNROLL, body, 0)

    g_wait(last % NB, last % NI)
    di_wait(last, last % NI)
    s_start(last % NB, last % NI)
    for c in range(last - 3, last + 1):
        s_wait(c % NB, c % NI)
    for ib in range(2):
        si_wait(last, ib)
        di_wait(last, ib)
    plsc.subcore_barrier()

    pltpu.sync_copy(agg_sh.at[pl.ds(stripe, ROWS_PER_TILE)],
                    out_hbm.at[pl.ds(stripe, ROWS_PER_TILE)])


def _tc_body(p_ref, x_ref, wl_ref, wr_ref, b_ref, o_ref, *, relu):
    acc = jnp.dot(p_ref[...], wl_ref[...], preferred_element_type=jnp.float32)
    acc += jnp.dot(x_ref[...], wr_ref[...], preferred_element_type=jnp.float32)
    acc += b_ref[...]
    o_ref[...] = jnp.maximum(acc, 0.0) if relu else acc


def _tc_combine(p, x, wlT, wrT, b, relu):
    blk = 2000
    grid = (N // blk,)
    row_spec = pl.BlockSpec((blk, D), lambda i: (i, 0))
    full_spec = pl.BlockSpec((D, D), lambda i: (0, 0))
    bias_spec = pl.BlockSpec((1, D), lambda i: (0, 0))
    return pl.pallas_call(
        functools.partial(_tc_body, relu=relu),
        grid=grid,
        in_specs=[row_spec, row_spec, full_spec, full_spec, bias_spec],
        out_specs=row_spec,
        out_shape=jax.ShapeDtypeStruct((N, D), jnp.float32),
    )(p, x, wlT, wrT, b.reshape(1, D))


def kernel(x, edge_index, Wl1, bl1, Wr1, Wl2, bl2, Wr2):
    src = edge_index[0]
    dst = edge_index[1]
    pad = E_PAD - E
    src_p = jnp.concatenate([src, jnp.zeros((pad,), jnp.int32)])
    dst_p = jnp.concatenate([dst, jnp.full((pad,), N, jnp.int32)])
    src_p = src_p.reshape(NS, CH_PER_TILE, CHUNK)
    dst_p = dst_p.reshape(NS, CH_PER_TILE, CHUNK)
    zeros = jnp.zeros((ROWS_PER_TILE, D), jnp.float32)

    p1 = _sc_aggregate(x, src_p, dst_p, zeros)
    h = _tc_combine(p1[:N], x, Wl1.T, Wr1.T, bl1, relu=True)
    p2 = _sc_aggregate(h, src_p, dst_p, zeros)
    return _tc_combine(p2[:N], h, Wl2.T, Wr2.T, bl2, relu=False)


# speedup vs baseline: 1.0001x; 1.0001x over previous
import functools

import jax
import jax.numpy as jnp
from jax import lax
from jax.experimental import pallas as pl
from jax.experimental.pallas import tpu as pltpu
from jax.experimental.pallas import tpu_sc as plsc

N = 10000
D = 128
E = 320000
NS = 16
CHUNK = 64
CH_PER_TILE = 320
E_PAD = NS * CH_PER_TILE * CHUNK
N_PAD = 10112
ROWS_PER_TILE = N_PAD // NS
NB = 4
NI = 8
LEAD = 4
UNROLL = 8


@functools.partial(
    pl.kernel,
    out_type=jax.ShapeDtypeStruct((N_PAD, D), jnp.float32),
    mesh=plsc.VectorSubcoreMesh(
        core_axis_name="c", subcore_axis_name="s", num_cores=1, num_subcores=NS
    ),
    scratch_types=[
        pltpu.VMEM((NB, CHUNK, D), jnp.float32),
        pltpu.VMEM((NI, CHUNK), jnp.int32),
        pltpu.VMEM((NI, CHUNK), jnp.int32),
        pltpu.VMEM_SHARED((N_PAD, D), jnp.float32),
        pltpu.SemaphoreType.DMA((NB,)),
        pltpu.SemaphoreType.DMA((NB,)),
        pltpu.SemaphoreType.DMA((NI,)),
        pltpu.SemaphoreType.DMA((NI,)),
    ],
)
def _sc_aggregate(x_hbm, src_hbm, dst_hbm, zeros_hbm, out_hbm,
                  rows_v, si_v, di_v, agg_sh, sg, ss, s_si, s_di):
    sid = lax.axis_index("s")
    stripe = sid * ROWS_PER_TILE
    last = CH_PER_TILE - 1

    pltpu.sync_copy(zeros_hbm, agg_sh.at[pl.ds(stripe, ROWS_PER_TILE)])
    plsc.subcore_barrier()

    def i_start(c, ib):
        pltpu.async_copy(src_hbm.at[sid, c], si_v.at[ib], s_si.at[ib])
        pltpu.async_copy(dst_hbm.at[sid, c], di_v.at[ib], s_di.at[ib])

    def si_wait(c, ib):
        pltpu.make_async_copy(src_hbm.at[sid, c], si_v.at[ib],
                              s_si.at[ib]).wait()

    def di_wait(c, ib):
        pltpu.make_async_copy(dst_hbm.at[sid, c], di_v.at[ib],
                              s_di.at[ib]).wait()

    def g_start(b, ib):
        pltpu.async_copy(x_hbm.at[si_v.at[ib]], rows_v.at[b], sg.at[b])

    def g_wait(b, ib):
        pltpu.make_async_copy(x_hbm.at[si_v.at[ib]], rows_v.at[b],
                              sg.at[b]).wait()

    def s_start(b, ib):
        pltpu.async_copy(rows_v.at[b], agg_sh.at[di_v.at[ib]], ss.at[b],
                         add=True)

    def s_wait(b, ib):
        pltpu.make_async_copy(rows_v.at[b], agg_sh.at[di_v.at[ib]],
                              ss.at[b]).wait()

    def step(j, k, skip):
        b, bm = k % NB, (k - 1) % NB
        ib, ibm, ibn = k % NI, (k - 1) % NI, (k + LEAD) % NI
        if skip <= -4:
            s_wait(b, (k - 4) % NI)
        i_start(jnp.minimum(j + LEAD, last), ibn)
        si_wait(j, ib)
        g_start(b, ib)
        if skip <= -1:
            g_wait(bm, ibm)
            di_wait(j - 1, ibm)
            s_start(bm, ibm)

    for c in range(LEAD):
        i_start(c, c % NI)
    for j in range(UNROLL):
        step(j, j, -j)

    def body(jj, carry):
        for k in range(UNROLL):
            step(jj * UNROLL + k, k, -4)
        return carry

    lax.fori_loop(1, CH_PER_TILE // UNROLL, body, 0)

    g_wait(last % NB, last % NI)
    di_wait(last, last % NI)
    s_start(last % NB, last % NI)
    for c in range(last - 3, last + 1):
        s_wait(c % NB, c % NI)
    for i in range(LEAD):
        si_wait(last, (last + 1 + i) % NI)
        di_wait(last, (last + 1 + i) % NI)
    plsc.subcore_barrier()

    pltpu.sync_copy(agg_sh.at[pl.ds(stripe, ROWS_PER_TILE)],
                    out_hbm.at[pl.ds(stripe, ROWS_PER_TILE)])


def _tc_body(p_ref, x_ref, wl_ref, wr_ref, b_ref, o_ref, *, relu):
    acc = jnp.dot(p_ref[...], wl_ref[...], preferred_element_type=jnp.float32)
    acc += jnp.dot(x_ref[...], wr_ref[...], preferred_element_type=jnp.float32)
    acc += b_ref[...]
    o_ref[...] = jnp.maximum(acc, 0.0) if relu else acc


def _tc_combine(p, x, wlT, wrT, b, relu):
    blk = 2000
    grid = (N // blk,)
    row_spec = pl.BlockSpec((blk, D), lambda i: (i, 0))
    full_spec = pl.BlockSpec((D, D), lambda i: (0, 0))
    bias_spec = pl.BlockSpec((1, D), lambda i: (0, 0))
    return pl.pallas_call(
        functools.partial(_tc_body, relu=relu),
        grid=grid,
        in_specs=[row_spec, row_spec, full_spec, full_spec, bias_spec],
        out_specs=row_spec,
        out_shape=jax.ShapeDtypeStruct((N, D), jnp.float32),
    )(p, x, wlT, wrT, b.reshape(1, D))


def kernel(x, edge_index, Wl1, bl1, Wr1, Wl2, bl2, Wr2):
    src = edge_index[0]
    dst = edge_index[1]
    pad = E_PAD - E
    src_p = jnp.concatenate([src, jnp.zeros((pad,), jnp.int32)])
    dst_p = jnp.concatenate([dst, jnp.full((pad,), N, jnp.int32)])
    src_p = src_p.reshape(NS, CH_PER_TILE, CHUNK)
    dst_p = dst_p.reshape(NS, CH_PER_TILE, CHUNK)
    zeros = jnp.zeros((ROWS_PER_TILE, D), jnp.float32)

    p1 = _sc_aggregate(x, src_p, dst_p, zeros)
    h = _tc_combine(p1[:N], x, Wl1.T, Wr1.T, bl1, relu=True)
    p2 = _sc_aggregate(h, src_p, dst_p, zeros)
    return _tc_combine(p2[:N], h, Wl2.T, Wr2.T, bl2, relu=False)
